# baseline (speedup 1.0000x reference)
import dataclasses
import functools

import jax
import jax.numpy as jnp
from jax import lax
from jax.experimental import pallas as pl
from jax.experimental.pallas import tpu as pltpu
from jax.experimental.pallas import tpu_sc as plsc

ALIGNED_H = 7
ALIGNED_W = 7
SPATIAL_SCALE = 0.25

_NC = 2
_NS = 16
_L = 16
_NW = _NC * _NS


@functools.cache
def _make_sc_blend(S, C2, G):
    ROWS = 4 * G
    WPW = S // (_NW * G)
    assert WPW % 4 == 0 and WPW >= 8
    mesh = plsc.VectorSubcoreMesh(core_axis_name="c", subcore_axis_name="s")
    cp = pltpu.CompilerParams()
    if "needs_layout_passes" in pltpu.CompilerParams.__dataclass_fields__:
        cp = dataclasses.replace(cp, needs_layout_passes=False)

    @functools.partial(
        pl.kernel,
        out_type=jax.ShapeDtypeStruct((S, C2), jnp.int32),
        mesh=mesh,
        compiler_params=cp,
        scratch_types=[
            pltpu.VMEM((4, ROWS), jnp.int32),
            pltpu.VMEM((2, ROWS, C2), jnp.int32),
            pltpu.VMEM((4, ROWS), jnp.int32),
            pltpu.VMEM((2, G, C2), jnp.int32),
        ] + [pltpu.SemaphoreType.DMA] * 12,
    )
    def sc_blend(table_hbm, idx_hbm, w_hbm, out_hbm,
                 idx_v, rows_v, w_v, out_v, *sems):
        isem, wsem, gsem, osem = sems[0:4], sems[4:8], sems[8:10], sems[10:12]
        wid = lax.axis_index("s") * _NC + lax.axis_index("c")

        def idx_copy(u, s):
            return pltpu.make_async_copy(
                idx_hbm.at[pl.ds((wid * WPW + u) * ROWS, ROWS)],
                idx_v.at[s], isem[s])

        def w_copy(u, s):
            return pltpu.make_async_copy(
                w_hbm.at[pl.ds((wid * WPW + u) * ROWS, ROWS)],
                w_v.at[s], wsem[s])

        def g_copy(isl, rsl):
            return pltpu.make_async_copy(
                table_hbm.at[idx_v.at[isl]], rows_v.at[rsl], gsem[rsl])

        def o_copy(u, s):
            return pltpu.make_async_copy(
                out_v.at[s], out_hbm.at[pl.ds((wid * WPW + u) * G, G)], osem[s])

        zv = lax.broadcasted_iota(jnp.int32, (_L,), 0) * 0

        def blend(rsl, wsl):
            @plsc.parallel_loop(0, G, step=1, unroll=2)
            def _samp(i):
                r = i * 4
                ws = [plsc.bitcast(
                          plsc.load_gather(w_v.at[wsl], [zv + (r + j)]),
                          jnp.bfloat16)
                      for j in range(4)]
                for cc in range(C2 // _L):
                    sl = pl.ds(cc * _L, _L)
                    a = (plsc.bitcast(rows_v[rsl, r, sl], jnp.bfloat16) * ws[0]
                         + plsc.bitcast(rows_v[rsl, r + 1, sl], jnp.bfloat16) * ws[1])
                    b = (plsc.bitcast(rows_v[rsl, r + 2, sl], jnp.bfloat16) * ws[2]
                         + plsc.bitcast(rows_v[rsl, r + 3, sl], jnp.bfloat16) * ws[3])
                    out_v[rsl, i, sl] = plsc.bitcast(a + b, jnp.int32)

        for h in range(4):
            idx_copy(h, h).start()
            w_copy(h, h).start()
        idx_copy(0, 0).wait()
        g_copy(0, 0).start()

        @pl.loop(0, WPW, step=4)
        def _win(t):
            for h in range(4):
                u = t + h
                rsl = h % 2
                g_copy(h, rsl).wait()
                @pl.when(u + 4 < WPW)
                def _():
                    idx_copy(u + 4, h).start()
                nsl = (h + 1) % 4
                if h == 3:
                    @pl.when(t + 4 < WPW)
                    def _():
                        idx_copy(0, nsl).wait()
                        w_copy(0, h).wait()
                        g_copy(nsl, (h + 1) % 2).start()
                    @pl.when(t + 4 >= WPW)
                    def _():
                        w_copy(0, h).wait()
                else:
                    idx_copy(0, nsl).wait()
                    w_copy(0, h).wait()
                    g_copy(nsl, (h + 1) % 2).start()
                if h < 2:
                    @pl.when(u >= 2)
                    def _():
                        o_copy(0, rsl).wait()
                else:
                    o_copy(0, rsl).wait()
                blend(rsl, h)
                o_copy(u, rsl).start()
                @pl.when(u + 4 < WPW)
                def _():
                    w_copy(u + 4, h).start()

        o_copy(0, 0).wait()
        o_copy(0, 1).wait()

    return sc_blend


def _prep(features, rois):
    B, C, H, W = features.shape
    N = rois.shape[0]
    AH, AW = ALIGNED_H, ALIGNED_W
    batch_idx = rois[:, 0].astype(jnp.int32)
    x1 = rois[:, 1] * SPATIAL_SCALE
    y1 = rois[:, 2] * SPATIAL_SCALE
    x2 = rois[:, 3] * SPATIAL_SCALE
    y2 = rois[:, 4] * SPATIAL_SCALE
    roi_w = jnp.maximum(x2 - x1, 0.0)
    roi_h = jnp.maximum(y2 - y1, 0.0)
    bin_w = roi_w / float(AW - 1)
    bin_h = roi_h / float(AH - 1)
    ph = jnp.arange(AH, dtype=jnp.float32)
    pw = jnp.arange(AW, dtype=jnp.float32)
    h = y1[:, None] + ph[None, :] * bin_h[:, None]
    w = x1[:, None] + pw[None, :] * bin_w[:, None]
    valid_h = (h >= 0) & (h < H)
    valid_w = (w >= 0) & (w < W)
    hs = jnp.minimum(jnp.floor(h), H - 2)
    ws = jnp.minimum(jnp.floor(w), W - 2)
    hs_i = jnp.clip(hs.astype(jnp.int32), 0, H - 2)
    ws_i = jnp.clip(ws.astype(jnp.int32), 0, W - 2)
    h_ratio = h - hs_i.astype(jnp.float32)
    w_ratio = w - ws_i.astype(jnp.float32)

    S = N * AH * AW
    valid = (valid_h[:, :, None] & valid_w[:, None, :]).reshape(S, 1)
    hr = jnp.broadcast_to(h_ratio[:, :, None], (N, AH, AW)).reshape(S, 1)
    wr = jnp.broadcast_to(w_ratio[:, None, :], (N, AH, AW)).reshape(S, 1)
    jj = jnp.arange(4, dtype=jnp.int32)[None, :]
    wa = jnp.where(jj >= 2, hr, 1.0 - hr)
    wb = jnp.where(jj % 2 == 1, wr, 1.0 - wr)
    w4 = jnp.where(valid, wa * wb, 0.0)
    tl = (batch_idx[:, None, None] * (H * W)
          + hs_i[:, :, None] * W + ws_i[:, None, :])
    off = jnp.array([0, 1, W, W + 1], jnp.int32)[None, :]
    idx4 = (tl.reshape(S, 1) + off).reshape(S * 4)
    return idx4, w4


def _to_bf16_bits(x):
    u = lax.bitcast_convert_type(x, jnp.uint32)
    return (u + 0x7FFF + ((u >> 16) & 1)) >> 16


def kernel(features, rois):
    B, C, H, W = features.shape
    N = rois.shape[0]
    AH, AW = ALIGNED_H, ALIGNED_W
    S = N * AH * AW
    G = 16
    assert S % (_NW * G * 4) == 0

    t = jnp.transpose(features, (0, 2, 3, 1)).reshape(B * H * W, C)
    lo = _to_bf16_bits(t[:, :C // 2])
    hi = _to_bf16_bits(t[:, C // 2:])
    table = ((hi << 16) | lo).astype(jnp.int32)

    idx4, w4 = _prep(features, rois)
    wr = _to_bf16_bits(w4)
    wi = ((wr << 16) | wr).astype(jnp.int32).reshape(S * 4)

    out = _make_sc_blend(S, C // 2, G)(table, idx4, wi)
    ou = lax.bitcast_convert_type(out, jnp.uint32)
    f_lo = lax.bitcast_convert_type(ou << 16, jnp.float32)
    f_hi = lax.bitcast_convert_type(ou & jnp.uint32(0xFFFF0000),
                                    jnp.float32)
    res = jnp.concatenate([f_lo, f_hi], axis=-1).reshape(N, AH, AW, C)
    return jnp.transpose(res, (0, 3, 1, 2))


# speedup vs baseline: 1.0273x; 1.0273x over previous
import dataclasses
import functools

import jax
import jax.numpy as jnp
from jax import lax
from jax.experimental import pallas as pl
from jax.experimental.pallas import tpu as pltpu
from jax.experimental.pallas import tpu_sc as plsc

ALIGNED_H = 7
ALIGNED_W = 7
SPATIAL_SCALE = 0.25

_NC = 2
_NS = 16
_L = 16
_NW = _NC * _NS


@functools.cache
def _make_sc_blend(S, C2, G):
    ROWS = 4 * G
    WPW = S // (_NW * G)
    assert WPW % 4 == 0 and WPW >= 8
    mesh = plsc.VectorSubcoreMesh(core_axis_name="c", subcore_axis_name="s")
    cp = pltpu.CompilerParams()
    if "needs_layout_passes" in pltpu.CompilerParams.__dataclass_fields__:
        cp = dataclasses.replace(cp, needs_layout_passes=False)
    if "use_tc_tiling_on_sc" in pltpu.CompilerParams.__dataclass_fields__:
        cp = dataclasses.replace(cp, use_tc_tiling_on_sc=True)

    @functools.partial(
        pl.kernel,
        out_type=jax.ShapeDtypeStruct((S, C2), jnp.int32),
        mesh=mesh,
        compiler_params=cp,
        scratch_types=[
            pltpu.VMEM((4, ROWS), jnp.int32),
            pltpu.VMEM((2, ROWS, C2), jnp.int32),
            pltpu.VMEM((4, ROWS), jnp.int32),
            pltpu.VMEM((2, G, C2), jnp.int32),
        ] + [pltpu.SemaphoreType.DMA] * 12,
    )
    def sc_blend(table_hbm, idx_hbm, w_hbm, out_hbm,
                 idx_v, rows_v, w_v, out_v, *sems):
        isem, wsem, gsem, osem = sems[0:4], sems[4:8], sems[8:10], sems[10:12]
        wid = lax.axis_index("s") * _NC + lax.axis_index("c")

        def idx_copy(u, s):
            return pltpu.make_async_copy(
                idx_hbm.at[pl.ds((wid * WPW + u) * ROWS, ROWS)],
                idx_v.at[s], isem[s])

        def w_copy(u, s):
            return pltpu.make_async_copy(
                w_hbm.at[pl.ds((wid * WPW + u) * ROWS, ROWS)],
                w_v.at[s], wsem[s])

        def g_copy(isl, rsl):
            return pltpu.make_async_copy(
                table_hbm.at[idx_v.at[isl]], rows_v.at[rsl], gsem[rsl])

        def o_copy(u, s):
            return pltpu.make_async_copy(
                out_v.at[s], out_hbm.at[pl.ds((wid * WPW + u) * G, G)], osem[s])

        zv = lax.broadcasted_iota(jnp.int32, (_L,), 0) * 0

        def blend(rsl, wsl):
            @plsc.parallel_loop(0, G, step=1, unroll=2)
            def _samp(i):
                r = i * 4
                ws = [plsc.bitcast(
                          plsc.load_gather(w_v.at[wsl], [zv + (r + j)]),
                          jnp.bfloat16)
                      for j in range(4)]
                for cc in range(C2 // _L):
                    sl = pl.ds(cc * _L, _L)
                    a = (plsc.bitcast(rows_v[rsl, r, sl], jnp.bfloat16) * ws[0]
                         + plsc.bitcast(rows_v[rsl, r + 1, sl], jnp.bfloat16) * ws[1])
                    b = (plsc.bitcast(rows_v[rsl, r + 2, sl], jnp.bfloat16) * ws[2]
                         + plsc.bitcast(rows_v[rsl, r + 3, sl], jnp.bfloat16) * ws[3])
                    out_v[rsl, i, sl] = plsc.bitcast(a + b, jnp.int32)

        for h in range(4):
            idx_copy(h, h).start()
            w_copy(h, h).start()
        idx_copy(0, 0).wait()
        g_copy(0, 0).start()

        @pl.loop(0, WPW, step=4)
        def _win(t):
            for h in range(4):
                u = t + h
                rsl = h % 2
                g_copy(h, rsl).wait()
                @pl.when(u + 4 < WPW)
                def _():
                    idx_copy(u + 4, h).start()
                nsl = (h + 1) % 4
                if h == 3:
                    @pl.when(t + 4 < WPW)
                    def _():
                        idx_copy(0, nsl).wait()
                        w_copy(0, h).wait()
                        g_copy(nsl, (h + 1) % 2).start()
                    @pl.when(t + 4 >= WPW)
                    def _():
                        w_copy(0, h).wait()
                else:
                    idx_copy(0, nsl).wait()
                    w_copy(0, h).wait()
                    g_copy(nsl, (h + 1) % 2).start()
                if h < 2:
                    @pl.when(u >= 2)
                    def _():
                        o_copy(0, rsl).wait()
                else:
                    o_copy(0, rsl).wait()
                blend(rsl, h)
                o_copy(u, rsl).start()
                @pl.when(u + 4 < WPW)
                def _():
                    w_copy(u + 4, h).start()

        o_copy(0, 0).wait()
        o_copy(0, 1).wait()

    return sc_blend


def _prep(features, rois):
    B, C, H, W = features.shape
    N = rois.shape[0]
    AH, AW = ALIGNED_H, ALIGNED_W
    batch_idx = rois[:, 0].astype(jnp.int32)
    x1 = rois[:, 1] * SPATIAL_SCALE
    y1 = rois[:, 2] * SPATIAL_SCALE
    x2 = rois[:, 3] * SPATIAL_SCALE
    y2 = rois[:, 4] * SPATIAL_SCALE
    roi_w = jnp.maximum(x2 - x1, 0.0)
    roi_h = jnp.maximum(y2 - y1, 0.0)
    bin_w = roi_w / float(AW - 1)
    bin_h = roi_h / float(AH - 1)
    ph = jnp.arange(AH, dtype=jnp.float32)
    pw = jnp.arange(AW, dtype=jnp.float32)
    h = y1[:, None] + ph[None, :] * bin_h[:, None]
    w = x1[:, None] + pw[None, :] * bin_w[:, None]
    valid_h = (h >= 0) & (h < H)
    valid_w = (w >= 0) & (w < W)
    hs = jnp.minimum(jnp.floor(h), H - 2)
    ws = jnp.minimum(jnp.floor(w), W - 2)
    hs_i = jnp.clip(hs.astype(jnp.int32), 0, H - 2)
    ws_i = jnp.clip(ws.astype(jnp.int32), 0, W - 2)
    h_ratio = h - hs_i.astype(jnp.float32)
    w_ratio = w - ws_i.astype(jnp.float32)

    valid = (valid_h[:, :, None] & valid_w[:, None, :]).astype(jnp.float32)
    hr = h_ratio[:, :, None]
    wr = w_ratio[:, None, :]
    w4 = jnp.stack(
        [(1.0 - hr) * (1.0 - wr) * valid,
         (1.0 - hr) * wr * valid,
         hr * (1.0 - wr) * valid,
         hr * wr * valid],
        axis=-1,
    ).reshape(N * AH * AW, 4)
    tl = (batch_idx[:, None, None] * (H * W)
          + hs_i[:, :, None] * W + ws_i[:, None, :])
    idx4 = jnp.stack([tl, tl + 1, tl + W, tl + W + 1], axis=-1)
    idx4 = idx4.reshape(N * AH * AW * 4).astype(jnp.int32)
    return idx4, w4


def _to_bf16_bits(x):
    u = lax.bitcast_convert_type(x, jnp.uint32)
    return (u + 0x7FFF + ((u >> 16) & 1)) >> 16


def kernel(features, rois):
    B, C, H, W = features.shape
    N = rois.shape[0]
    AH, AW = ALIGNED_H, ALIGNED_W
    S = N * AH * AW
    G = 16
    assert S % (_NW * G * 4) == 0

    t = jnp.transpose(features, (0, 2, 3, 1)).reshape(B * H * W, C)
    lo = _to_bf16_bits(t[:, :C // 2])
    hi = _to_bf16_bits(t[:, C // 2:])
    table = ((hi << 16) | lo).astype(jnp.int32)

    idx4, w4 = _prep(features, rois)
    wr = _to_bf16_bits(w4)
    wi = ((wr << 16) | wr).astype(jnp.int32).reshape(S * 4)

    out = _make_sc_blend(S, C // 2, G)(table, idx4, wi)
    ou = lax.bitcast_convert_type(out, jnp.uint32)
    f_lo = lax.bitcast_convert_type(ou << 16, jnp.float32)
    f_hi = lax.bitcast_convert_type(ou & jnp.uint32(0xFFFF0000),
                                    jnp.float32)
    res = jnp.concatenate([f_lo, f_hi], axis=-1).reshape(N, AH, AW, C)
    return jnp.transpose(res, (0, 3, 1, 2))


# speedup vs baseline: 1.0378x; 1.0102x over previous
import dataclasses
import functools

import jax
import jax.numpy as jnp
from jax import lax
from jax.experimental import pallas as pl
from jax.experimental.pallas import tpu as pltpu
from jax.experimental.pallas import tpu_sc as plsc

ALIGNED_H = 7
ALIGNED_W = 7
SPATIAL_SCALE = 0.25

_NC = 2
_NS = 16
_L = 16
_NW = _NC * _NS


@functools.cache
def _make_sc_blend(S, C2, G):
    ROWS = 4 * G
    WPW = S // (_NW * G)
    assert WPW % 2 == 0 and WPW >= 6
    mesh = plsc.VectorSubcoreMesh(core_axis_name="c", subcore_axis_name="s")
    cp = pltpu.CompilerParams()
    if "needs_layout_passes" in pltpu.CompilerParams.__dataclass_fields__:
        cp = dataclasses.replace(cp, needs_layout_passes=False)

    @functools.partial(
        pl.kernel,
        out_type=jax.ShapeDtypeStruct((S, C2), jnp.int32),
        mesh=mesh,
        compiler_params=cp,
        scratch_types=[
            pltpu.VMEM((2, ROWS), jnp.int32),
            pltpu.VMEM((2, ROWS, C2), jnp.int32),
            pltpu.VMEM((2, ROWS), jnp.int32),
            pltpu.VMEM((2, G, C2), jnp.int32),
        ] + [pltpu.SemaphoreType.DMA] * 8,
    )
    def sc_blend(table_hbm, idx_hbm, w_hbm, out_hbm,
                 idx_v, rows_v, w_v, out_v, *sems):
        isem, wsem, gsem, osem = sems[0:2], sems[2:4], sems[4:6], sems[6:8]
        wid = lax.axis_index("s") * _NC + lax.axis_index("c")

        def idx_copy(u, s):
            return pltpu.make_async_copy(
                idx_hbm.at[pl.ds((wid * WPW + u) * ROWS, ROWS)],
                idx_v.at[s], isem[s])

        def w_copy(u, s):
            return pltpu.make_async_copy(
                w_hbm.at[pl.ds((wid * WPW + u) * ROWS, ROWS)],
                w_v.at[s], wsem[s])

        def g_copy(isl, rsl):
            return pltpu.make_async_copy(
                table_hbm.at[idx_v.at[isl]], rows_v.at[rsl], gsem[rsl])

        def o_copy(u, s):
            return pltpu.make_async_copy(
                out_v.at[s], out_hbm.at[pl.ds((wid * WPW + u) * G, G)], osem[s])

        zv = lax.broadcasted_iota(jnp.int32, (_L,), 0) * 0

        def blend(rsl, wsl):
            @plsc.parallel_loop(0, G, step=1, unroll=2)
            def _samp(i):
                r = i * 4
                ws = [plsc.bitcast(
                          plsc.load_gather(w_v.at[wsl], [zv + (r + j)]),
                          jnp.bfloat16)
                      for j in range(4)]
                for cc in range(C2 // _L):
                    sl = pl.ds(cc * _L, _L)
                    a = (plsc.bitcast(rows_v[rsl, r, sl], jnp.bfloat16) * ws[0]
                         + plsc.bitcast(rows_v[rsl, r + 1, sl], jnp.bfloat16) * ws[1])
                    b = (plsc.bitcast(rows_v[rsl, r + 2, sl], jnp.bfloat16) * ws[2]
                         + plsc.bitcast(rows_v[rsl, r + 3, sl], jnp.bfloat16) * ws[3])
                    out_v[rsl, i, sl] = plsc.bitcast(a + b, jnp.int32)

        for h in range(2):
            idx_copy(h, h).start()
            w_copy(h, h).start()
        idx_copy(0, 0).wait()
        g_copy(0, 0).start()

        @pl.loop(0, WPW, step=2)
        def _win(t):
            for h in range(2):
                u = t + h
                o = h ^ 1
                g_copy(h, h).wait()
                @pl.when(u + 2 < WPW)
                def _():
                    idx_copy(u + 2, h).start()
                if h == 1:
                    @pl.when(t + 2 < WPW)
                    def _():
                        idx_copy(0, o).wait()
                        g_copy(o, o).start()
                else:
                    idx_copy(0, o).wait()
                    g_copy(o, o).start()
                w_copy(0, h).wait()
                @pl.when(u >= 2)
                def _():
                    o_copy(0, h).wait()
                blend(h, h)
                o_copy(u, h).start()
                @pl.when(u + 2 < WPW)
                def _():
                    w_copy(u + 2, h).start()

        o_copy(0, 0).wait()
        o_copy(0, 1).wait()

    return sc_blend


def _prep(features, rois):
    B, C, H, W = features.shape
    N = rois.shape[0]
    AH, AW = ALIGNED_H, ALIGNED_W
    batch_idx = rois[:, 0].astype(jnp.int32)
    x1 = rois[:, 1] * SPATIAL_SCALE
    y1 = rois[:, 2] * SPATIAL_SCALE
    x2 = rois[:, 3] * SPATIAL_SCALE
    y2 = rois[:, 4] * SPATIAL_SCALE
    roi_w = jnp.maximum(x2 - x1, 0.0)
    roi_h = jnp.maximum(y2 - y1, 0.0)
    bin_w = roi_w / float(AW - 1)
    bin_h = roi_h / float(AH - 1)
    ph = jnp.arange(AH, dtype=jnp.float32)
    pw = jnp.arange(AW, dtype=jnp.float32)
    h = y1[:, None] + ph[None, :] * bin_h[:, None]
    w = x1[:, None] + pw[None, :] * bin_w[:, None]
    valid_h = (h >= 0) & (h < H)
    valid_w = (w >= 0) & (w < W)
    hs = jnp.minimum(jnp.floor(h), H - 2)
    ws = jnp.minimum(jnp.floor(w), W - 2)
    hs_i = jnp.clip(hs.astype(jnp.int32), 0, H - 2)
    ws_i = jnp.clip(ws.astype(jnp.int32), 0, W - 2)
    h_ratio = h - hs_i.astype(jnp.float32)
    w_ratio = w - ws_i.astype(jnp.float32)

    valid = (valid_h[:, :, None] & valid_w[:, None, :]).astype(jnp.float32)
    hr = h_ratio[:, :, None]
    wr = w_ratio[:, None, :]
    w4 = jnp.stack(
        [(1.0 - hr) * (1.0 - wr) * valid,
         (1.0 - hr) * wr * valid,
         hr * (1.0 - wr) * valid,
         hr * wr * valid],
        axis=-1,
    ).reshape(N * AH * AW, 4)
    tl = (batch_idx[:, None, None] * (H * W)
          + hs_i[:, :, None] * W + ws_i[:, None, :])
    idx4 = jnp.stack([tl, tl + 1, tl + W, tl + W + 1], axis=-1)
    idx4 = idx4.reshape(N * AH * AW * 4).astype(jnp.int32)
    return idx4, w4


def _to_bf16_bits(x):
    u = lax.bitcast_convert_type(x, jnp.uint32)
    return (u + 0x7FFF + ((u >> 16) & 1)) >> 16


def kernel(features, rois):
    B, C, H, W = features.shape
    N = rois.shape[0]
    AH, AW = ALIGNED_H, ALIGNED_W
    S = N * AH * AW
    G = 16
    NCHUNK = 2
    assert S % (NCHUNK * _NW * G * 2) == 0 and N % NCHUNK == 0

    t = jnp.transpose(features, (0, 2, 3, 1)).reshape(B * H * W, C)
    lo = _to_bf16_bits(t[:, :C // 2])
    hi = _to_bf16_bits(t[:, C // 2:])
    table = ((hi << 16) | lo).astype(jnp.int32)

    idx4, w4 = _prep(features, rois)
    wr = _to_bf16_bits(w4)
    wi = ((wr << 16) | wr).astype(jnp.int32).reshape(S * 4)

    Sc, Nc = S // NCHUNK, N // NCHUNK
    parts = []
    for k in range(NCHUNK):
        out = _make_sc_blend(Sc, C // 2, G)(
            table,
            lax.slice_in_dim(idx4, k * Sc * 4, (k + 1) * Sc * 4),
            lax.slice_in_dim(wi, k * Sc * 4, (k + 1) * Sc * 4))
        ou = lax.bitcast_convert_type(out, jnp.uint32)
        f_lo = lax.bitcast_convert_type(ou << 16, jnp.float32)
        f_hi = lax.bitcast_convert_type(ou & jnp.uint32(0xFFFF0000),
                                        jnp.float32)
        res = jnp.concatenate([f_lo, f_hi], axis=-1)
        res = res.reshape(Nc, AH, AW, C)
        parts.append(jnp.transpose(res, (0, 3, 1, 2)))
    return jnp.concatenate(parts, axis=0)


# speedup vs baseline: 1.1050x; 1.0648x over previous
import dataclasses
import functools

import jax
import jax.numpy as jnp
from jax import lax
from jax.experimental import pallas as pl
from jax.experimental.pallas import tpu as pltpu
from jax.experimental.pallas import tpu_sc as plsc

ALIGNED_H = 7
ALIGNED_W = 7
SPATIAL_SCALE = 0.25

_NC = 2
_NS = 16
_L = 16
_NW = _NC * _NS


@functools.cache
def _make_sc_blend(S, C2, G, W_IMG):
    ROWS = 4 * G
    WPW = S // (_NW * G)
    assert WPW % 2 == 0 and WPW >= 6
    mesh = plsc.VectorSubcoreMesh(core_axis_name="c", subcore_axis_name="s")
    cp = pltpu.CompilerParams()
    if "needs_layout_passes" in pltpu.CompilerParams.__dataclass_fields__:
        cp = dataclasses.replace(cp, needs_layout_passes=False)

    @functools.partial(
        pl.kernel,
        out_type=jax.ShapeDtypeStruct((S, C2), jnp.int32),
        mesh=mesh,
        compiler_params=cp,
        scratch_types=[
            pltpu.VMEM((2, G), jnp.int32),
            pltpu.VMEM((2, ROWS), jnp.int32),
            pltpu.VMEM((2, ROWS, C2), jnp.int32),
            pltpu.VMEM((2, 4, G), jnp.int32),
            pltpu.VMEM((2, G, C2), jnp.int32),
        ] + [pltpu.SemaphoreType.DMA] * 8,
    )
    def sc_blend(table_hbm, idx_hbm, w_hbm, out_hbm,
                 tl_v, idx_v, rows_v, w_v, out_v, *sems):
        isem, wsem, gsem, osem = sems[0:2], sems[2:4], sems[4:6], sems[6:8]
        wid = lax.axis_index("s") * _NC + lax.axis_index("c")

        def idx_copy(u, s):
            return pltpu.make_async_copy(
                idx_hbm.at[pl.ds((wid * WPW + u) * G, G)], tl_v.at[s], isem[s])

        def w_copy(u, s):
            return pltpu.make_async_copy(
                w_hbm.at[wid * WPW + u], w_v.at[s], wsem[s])

        def g_copy(isl, rsl):
            return pltpu.make_async_copy(
                table_hbm.at[idx_v.at[isl]], rows_v.at[rsl], gsem[rsl])

        def o_copy(u, s):
            return pltpu.make_async_copy(
                out_v.at[s], out_hbm.at[pl.ds((wid * WPW + u) * G, G)], osem[s])

        iv = lax.broadcasted_iota(jnp.int32, (_L,), 0)
        zv = iv * 0
        OFF = (0, 1, W_IMG, W_IMG + 1)

        def expand_idx(s):
            tlc = tl_v[s, pl.ds(0, _L)]
            for j in range(4):
                plsc.store_scatter(idx_v.at[s], [iv * 4 + j], tlc + OFF[j])

        def blend(rsl, wsl):
            @plsc.parallel_loop(0, G, step=1, unroll=2)
            def _samp(i):
                r = i * 4
                ws = [plsc.bitcast(
                          plsc.load_gather(w_v.at[wsl], [zv + j, zv + i]),
                          jnp.bfloat16)
                      for j in range(4)]
                for cc in range(C2 // _L):
                    sl = pl.ds(cc * _L, _L)
                    a = (plsc.bitcast(rows_v[rsl, r, sl], jnp.bfloat16) * ws[0]
                         + plsc.bitcast(rows_v[rsl, r + 1, sl], jnp.bfloat16) * ws[1])
                    b = (plsc.bitcast(rows_v[rsl, r + 2, sl], jnp.bfloat16) * ws[2]
                         + plsc.bitcast(rows_v[rsl, r + 3, sl], jnp.bfloat16) * ws[3])
                    out_v[rsl, i, sl] = plsc.bitcast(a + b, jnp.int32)

        for h in range(2):
            idx_copy(h, h).start()
            w_copy(h, h).start()
        idx_copy(0, 0).wait()
        expand_idx(0)
        g_copy(0, 0).start()

        @pl.loop(0, WPW, step=2)
        def _win(t):
            for h in range(2):
                u = t + h
                o = h ^ 1
                g_copy(h, h).wait()
                @pl.when(u + 2 < WPW)
                def _():
                    idx_copy(u + 2, h).start()
                if h == 1:
                    @pl.when(t + 2 < WPW)
                    def _():
                        idx_copy(0, o).wait()
                        expand_idx(o)
                        g_copy(o, o).start()
                else:
                    idx_copy(0, o).wait()
                    expand_idx(o)
                    g_copy(o, o).start()
                w_copy(0, h).wait()
                @pl.when(u >= 2)
                def _():
                    o_copy(0, h).wait()
                blend(h, h)
                o_copy(u, h).start()
                @pl.when(u + 2 < WPW)
                def _():
                    w_copy(u + 2, h).start()

        o_copy(0, 0).wait()
        o_copy(0, 1).wait()

    return sc_blend


def _prep(features, rois):
    B, C, H, W = features.shape
    N = rois.shape[0]
    AH, AW = ALIGNED_H, ALIGNED_W
    batch_idx = rois[:, 0].astype(jnp.int32)
    x1 = rois[:, 1] * SPATIAL_SCALE
    y1 = rois[:, 2] * SPATIAL_SCALE
    x2 = rois[:, 3] * SPATIAL_SCALE
    y2 = rois[:, 4] * SPATIAL_SCALE
    roi_w = jnp.maximum(x2 - x1, 0.0)
    roi_h = jnp.maximum(y2 - y1, 0.0)
    bin_w = roi_w / float(AW - 1)
    bin_h = roi_h / float(AH - 1)
    ph = jnp.arange(AH, dtype=jnp.float32)
    pw = jnp.arange(AW, dtype=jnp.float32)
    h = y1[:, None] + ph[None, :] * bin_h[:, None]
    w = x1[:, None] + pw[None, :] * bin_w[:, None]
    valid_h = (h >= 0) & (h < H)
    valid_w = (w >= 0) & (w < W)
    hs = jnp.minimum(jnp.floor(h), H - 2)
    ws = jnp.minimum(jnp.floor(w), W - 2)
    hs_i = jnp.clip(hs.astype(jnp.int32), 0, H - 2)
    ws_i = jnp.clip(ws.astype(jnp.int32), 0, W - 2)
    h_ratio = h - hs_i.astype(jnp.float32)
    w_ratio = w - ws_i.astype(jnp.float32)

    S = N * AH * AW
    valid = (valid_h[:, :, None] & valid_w[:, None, :]).astype(jnp.float32)
    hr = h_ratio[:, :, None]
    wr = w_ratio[:, None, :]
    w_pl = jnp.stack(
        [((1.0 - hr) * (1.0 - wr) * valid).reshape(S),
         ((1.0 - hr) * wr * valid).reshape(S),
         (hr * (1.0 - wr) * valid).reshape(S),
         (hr * wr * valid).reshape(S)],
        axis=0,
    )
    tl = (batch_idx[:, None, None] * (H * W)
          + hs_i[:, :, None] * W + ws_i[:, None, :])
    return tl.reshape(S).astype(jnp.int32), w_pl


def _to_bf16_bits(x):
    u = lax.bitcast_convert_type(x, jnp.uint32)
    return (u + 0x7FFF + ((u >> 16) & 1)) >> 16


def kernel(features, rois):
    B, C, H, W = features.shape
    N = rois.shape[0]
    AH, AW = ALIGNED_H, ALIGNED_W
    S = N * AH * AW
    G = 16
    NCHUNK = 2
    assert S % (NCHUNK * _NW * G * 2) == 0 and N % NCHUNK == 0

    t = jnp.transpose(features, (0, 2, 3, 1)).reshape(B * H * W, C)
    lo = _to_bf16_bits(t[:, :C // 2])
    hi = _to_bf16_bits(t[:, C // 2:])
    table = ((hi << 16) | lo).astype(jnp.int32)

    tl, w_pl = _prep(features, rois)
    wr = _to_bf16_bits(w_pl)
    wi = ((wr << 16) | wr).astype(jnp.int32)
    wib = wi.reshape(4, S // G, G).transpose(1, 0, 2)

    Sc, Nc = S // NCHUNK, N // NCHUNK
    parts = []
    for k in range(NCHUNK):
        out = _make_sc_blend(Sc, C // 2, G, W)(
            table,
            lax.slice_in_dim(tl, k * Sc, (k + 1) * Sc),
            lax.slice_in_dim(wib, k * (Sc // G), (k + 1) * (Sc // G)))
        ou = lax.bitcast_convert_type(out, jnp.uint32)
        f_lo = lax.bitcast_convert_type(ou << 16, jnp.float32)
        f_hi = lax.bitcast_convert_type(ou & jnp.uint32(0xFFFF0000),
                                        jnp.float32)
        res = jnp.concatenate([f_lo, f_hi], axis=-1)
        res = res.reshape(Nc, AH, AW, C)
        parts.append(jnp.transpose(res, (0, 3, 1, 2)))
    return jnp.concatenate(parts, axis=0)
